# baseline (speedup 1.0000x reference)
import jax
import jax.numpy as jnp
from jax.experimental import pallas as pl
from jax.experimental.pallas import tpu as pltpu
from jax.experimental.pallas import tpu_sc as plsc

_NH = 16
_NB = 32
_Q = 2048
_K = 2048
_THR = (8, 12, 16, 23, 32, 46, 64, 91)

_NP = 136
_LAW = 4096
_EXT = 4352
_NHW = 2
_QSL = 512
_BLK = 8
_NBLK = _QSL // _BLK


_RPS = 8


def _la_kernel(table_ref, la_ref, lext_ref, u_ref, sems):
    g = pl.program_id(0)
    par = jax.lax.rem(g, 2)

    @pl.when(g == 0)
    def _():
        j = jax.lax.broadcasted_iota(jnp.int32, (1, _EXT), 1)
        d = j - 128 - (_Q - 1)
        a = jnp.abs(d)
        v = jnp.minimum(a, 7)
        for t in _THR:
            v = v + (a >= t).astype(jnp.int32)
        bucket = jnp.where(d > 0, 16, 0) + v
        acc = jnp.zeros((_NH, _EXT), jnp.float32)
        for b in range(_NB):
            col = table_ref[b, :].reshape(_NH, 1)
            acc = jnp.where(bucket == b, col, acc)
        lext_ref[...] = acc

    def cp(par_, i, g_):
        slot = par_ * _RPS + i
        return pltpu.make_async_copy(
            u_ref.at[slot, :, pl.ds(0, _LAW)],
            la_ref.at[:, g_ * _RPS + i, :],
            sems.at[slot],
        )

    @pl.when(g >= 2)
    def _():
        for i in range(_RPS):
            cp(par, i, g - 2).wait()

    for i in range(_RPS):
        p = g * _RPS + i
        shift = jax.lax.rem(jnp.int32(_EXT - 135) + p, jnp.int32(_EXT))
        u_ref[par * _RPS + i] = pltpu.roll(lext_ref[...], shift, 1)
        cp(par, i, g).start()

    @pl.when(g == _NP // _RPS - 1)
    def _():
        for i in range(_RPS):
            cp(1 - par, i, g - 1).wait()
        for i in range(_RPS):
            cp(par, i, g).wait()


def _build_la(table):
    return pl.pallas_call(
        _la_kernel,
        grid=(_NP // _RPS,),
        in_specs=[pl.BlockSpec((_NB, _NH), lambda g: (0, 0))],
        out_specs=pl.BlockSpec(memory_space=pl.ANY),
        out_shape=jax.ShapeDtypeStruct((_NH, _NP, _LAW), jnp.float32),
        scratch_shapes=[
            pltpu.VMEM((_NH, _EXT), jnp.float32),
            pltpu.VMEM((2 * _RPS, _NH, _EXT), jnp.float32),
            pltpu.SemaphoreType.DMA((2 * _RPS,)),
        ],
    )(table)


def _sc_expand(la):
    @pl.kernel(
        out_type=jax.ShapeDtypeStruct((_NH, _Q, _K), jnp.float32),
        mesh=plsc.VectorSubcoreMesh(core_axis_name="c", subcore_axis_name="s"),
        scratch_types=[
            pltpu.VMEM((3, _NHW, _BLK, _K), jnp.float32),
            pltpu.SemaphoreType.DMA((3,)),
            pltpu.SemaphoreType.DMA((3,)),
        ],
        compiler_params=pltpu.CompilerParams(use_tc_tiling_on_sc=True),
    )
    def body(la_hbm, out_hbm, st_ref, sst, sout):
        w = jax.lax.axis_index("c") * 16 + jax.lax.axis_index("s")
        h0 = jax.lax.div(w, 4) * _NHW
        q0 = jax.lax.rem(w, 4) * _QSL

        def stage_cp(buf, qb):
            s0 = (_Q - 1) - qb
            c2 = jax.lax.div(s0 + 120, 128)
            p0x8 = jax.lax.div(c2 * 128 + 7 - s0, 8) * 8
            return pltpu.make_async_copy(
                la_hbm.at[pl.ds(h0, _NHW), pl.ds(p0x8, _BLK),
                          pl.ds(c2 * 128, _K)],
                st_ref.at[buf],
                sst.at[buf],
            )

        def out_cp(buf, qb):
            return pltpu.make_async_copy(
                st_ref.at[buf],
                out_hbm.at[pl.ds(h0, _NHW), pl.ds(qb, _BLK), :],
                sout.at[buf],
            )

        def qb_of(bb):
            return (jax.lax.div(q0, _BLK) + bb) * _BLK

        stage_cp(jnp.int32(0), qb_of(jnp.int32(0))).start()

        @pl.loop(0, _NBLK)
        def _(b):
            buf = jax.lax.rem(b, 3)
            bufn = jax.lax.rem(b + 1, 3)

            @pl.when(b >= 2)
            def _():
                out_cp(bufn, qb_of(b - 2)).wait()

            @pl.when(b + 1 < _NBLK)
            def _():
                stage_cp(bufn, qb_of(b + 1)).start()

            stage_cp(buf, qb_of(b)).wait()
            out_cp(buf, qb_of(b)).start()

        for b in (_NBLK - 2, _NBLK - 1):
            out_cp(jnp.int32(b % 3), qb_of(jnp.int32(b))).wait()

    return body(la)


def kernel(x, table):
    del x
    return _sc_expand(_build_la(table))


# speedup vs baseline: 1.0021x; 1.0021x over previous
import jax
import jax.numpy as jnp
from jax.experimental import pallas as pl
from jax.experimental.pallas import tpu as pltpu
from jax.experimental.pallas import tpu_sc as plsc

_NH = 16
_NB = 32
_Q = 2048
_K = 2048
_THR = (8, 12, 16, 23, 32, 46, 64, 91)

_NP = 136
_LAW = 4096
_EXT = 4352
_NHW = 2
_QSL = 512
_BLK = 8
_NBLK = _QSL // _BLK


_RPS = 4


def _la_kernel(table_ref, la_ref, lext_ref, u_ref, sems):
    g = pl.program_id(0)
    par = jax.lax.rem(g, 2)

    @pl.when(g == 0)
    def _():
        j = jax.lax.broadcasted_iota(jnp.int32, (1, _EXT), 1)
        d = j - 128 - (_Q - 1)
        a = jnp.abs(d)
        v = jnp.minimum(a, 7)
        for t in _THR:
            v = v + (a >= t).astype(jnp.int32)
        bucket = jnp.where(d > 0, 16, 0) + v
        acc = jnp.zeros((_NH, _EXT), jnp.float32)
        for b in range(_NB):
            col = table_ref[b, :].reshape(_NH, 1)
            acc = jnp.where(bucket == b, col, acc)
        lext_ref[...] = acc

    def cp(par_, i, g_):
        slot = par_ * _RPS + i
        return pltpu.make_async_copy(
            u_ref.at[slot, :, pl.ds(0, _LAW)],
            la_ref.at[:, g_ * _RPS + i, :],
            sems.at[slot],
        )

    @pl.when(g >= 2)
    def _():
        for i in range(_RPS):
            cp(par, i, g - 2).wait()

    for i in range(_RPS):
        p = g * _RPS + i
        shift = jax.lax.rem(jnp.int32(_EXT - 135) + p, jnp.int32(_EXT))
        u_ref[par * _RPS + i] = pltpu.roll(lext_ref[...], shift, 1)
        cp(par, i, g).start()

    @pl.when(g == _NP // _RPS - 1)
    def _():
        for i in range(_RPS):
            cp(1 - par, i, g - 1).wait()
        for i in range(_RPS):
            cp(par, i, g).wait()


def _build_la(table):
    return pl.pallas_call(
        _la_kernel,
        grid=(_NP // _RPS,),
        in_specs=[pl.BlockSpec((_NB, _NH), lambda g: (0, 0))],
        out_specs=pl.BlockSpec(memory_space=pl.ANY),
        out_shape=jax.ShapeDtypeStruct((_NH, _NP, _LAW), jnp.float32),
        scratch_shapes=[
            pltpu.VMEM((_NH, _EXT), jnp.float32),
            pltpu.VMEM((2 * _RPS, _NH, _EXT), jnp.float32),
            pltpu.SemaphoreType.DMA((2 * _RPS,)),
        ],
    )(table)


def _sc_expand(la):
    @pl.kernel(
        out_type=jax.ShapeDtypeStruct((_NH, _Q, _K), jnp.float32),
        mesh=plsc.VectorSubcoreMesh(core_axis_name="c", subcore_axis_name="s"),
        scratch_types=[
            pltpu.VMEM((3, _NHW, _BLK, _K), jnp.float32),
            pltpu.SemaphoreType.DMA((3,)),
            pltpu.SemaphoreType.DMA((3,)),
        ],
        compiler_params=pltpu.CompilerParams(use_tc_tiling_on_sc=True),
    )
    def body(la_hbm, out_hbm, st_ref, sst, sout):
        w = jax.lax.axis_index("c") * 16 + jax.lax.axis_index("s")
        h0 = jax.lax.div(w, 4) * _NHW
        q0 = jax.lax.rem(w, 4) * _QSL

        def stage_cp(buf, qb):
            s0 = (_Q - 1) - qb
            c2 = jax.lax.div(s0 + 120, 128)
            p0x8 = jax.lax.div(c2 * 128 + 7 - s0, 8) * 8
            return pltpu.make_async_copy(
                la_hbm.at[pl.ds(h0, _NHW), pl.ds(p0x8, _BLK),
                          pl.ds(c2 * 128, _K)],
                st_ref.at[buf],
                sst.at[buf],
            )

        def out_cp(buf, qb):
            return pltpu.make_async_copy(
                st_ref.at[buf],
                out_hbm.at[pl.ds(h0, _NHW), pl.ds(qb, _BLK), :],
                sout.at[buf],
            )

        def qb_of(bb):
            return (jax.lax.div(q0, _BLK) + bb) * _BLK

        stage_cp(jnp.int32(0), qb_of(jnp.int32(0))).start()

        @pl.loop(0, _NBLK)
        def _(b):
            buf = jax.lax.rem(b, 3)
            bufn = jax.lax.rem(b + 1, 3)

            @pl.when(b >= 2)
            def _():
                out_cp(bufn, qb_of(b - 2)).wait()

            @pl.when(b + 1 < _NBLK)
            def _():
                stage_cp(bufn, qb_of(b + 1)).start()

            stage_cp(buf, qb_of(b)).wait()
            out_cp(buf, qb_of(b)).start()

        for b in (_NBLK - 2, _NBLK - 1):
            out_cp(jnp.int32(b % 3), qb_of(jnp.int32(b))).wait()

    return body(la)


def kernel(x, table):
    del x
    return _sc_expand(_build_la(table))


# speedup vs baseline: 1.0070x; 1.0049x over previous
import jax
import jax.numpy as jnp
from jax.experimental import pallas as pl
from jax.experimental.pallas import tpu as pltpu
from jax.experimental.pallas import tpu_sc as plsc

_NH = 16
_NB = 32
_Q = 2048
_K = 2048
_THR = (8, 12, 16, 23, 32, 46, 64, 91)

_NP = 144
_LAW = 4096
_EXT = 4352
_NHW = 1
_QSL = 1024
_BLK = 16
_NBLK = _QSL // _BLK


_RPS = 4


def _la_kernel(table_ref, la_ref, lext_ref, u_ref, sems):
    g = pl.program_id(0)
    par = jax.lax.rem(g, 2)

    @pl.when(g == 0)
    def _():
        j = jax.lax.broadcasted_iota(jnp.int32, (1, _EXT), 1)
        d = j - 128 - (_Q - 1)
        a = jnp.abs(d)
        v = jnp.minimum(a, 7)
        for t in _THR:
            v = v + (a >= t).astype(jnp.int32)
        bucket = jnp.where(d > 0, 16, 0) + v
        acc = jnp.zeros((_NH, _EXT), jnp.float32)
        for b in range(_NB):
            col = table_ref[b, :].reshape(_NH, 1)
            acc = jnp.where(bucket == b, col, acc)
        lext_ref[...] = acc

    def cp(par_, i, g_):
        slot = par_ * _RPS + i
        return pltpu.make_async_copy(
            u_ref.at[slot, :, pl.ds(0, _LAW)],
            la_ref.at[:, g_ * _RPS + i, :],
            sems.at[slot],
        )

    @pl.when(g >= 2)
    def _():
        for i in range(_RPS):
            cp(par, i, g - 2).wait()

    for i in range(_RPS):
        p = g * _RPS + i
        shift = jax.lax.rem(jnp.int32(_EXT - 135) + p, jnp.int32(_EXT))
        u_ref[par * _RPS + i] = pltpu.roll(lext_ref[...], shift, 1)
        cp(par, i, g).start()

    @pl.when(g == _NP // _RPS - 1)
    def _():
        for i in range(_RPS):
            cp(1 - par, i, g - 1).wait()
        for i in range(_RPS):
            cp(par, i, g).wait()


def _build_la(table):
    return pl.pallas_call(
        _la_kernel,
        grid=(_NP // _RPS,),
        in_specs=[pl.BlockSpec((_NB, _NH), lambda g: (0, 0))],
        out_specs=pl.BlockSpec(memory_space=pl.ANY),
        out_shape=jax.ShapeDtypeStruct((_NH, _NP, _LAW), jnp.float32),
        scratch_shapes=[
            pltpu.VMEM((_NH, _EXT), jnp.float32),
            pltpu.VMEM((2 * _RPS, _NH, _EXT), jnp.float32),
            pltpu.SemaphoreType.DMA((2 * _RPS,)),
        ],
    )(table)


def _sc_expand(la):
    @pl.kernel(
        out_type=jax.ShapeDtypeStruct((_NH, _Q, _K), jnp.float32),
        mesh=plsc.VectorSubcoreMesh(core_axis_name="c", subcore_axis_name="s"),
        scratch_types=[
            pltpu.VMEM((3, _NHW, _BLK, _K), jnp.float32),
            pltpu.SemaphoreType.DMA((3,)),
            pltpu.SemaphoreType.DMA((3,)),
        ],
        compiler_params=pltpu.CompilerParams(use_tc_tiling_on_sc=True),
    )
    def body(la_hbm, out_hbm, st_ref, sst, sout):
        w = jax.lax.axis_index("c") * 16 + jax.lax.axis_index("s")
        h0 = jax.lax.div(w, 2) * _NHW
        q0 = jax.lax.rem(w, 2) * _QSL

        def stage_cp(buf, qb):
            s0 = (_Q - 1) - qb
            c2 = jax.lax.div(s0 + 120, 128)
            p0x8 = jax.lax.div(c2 * 128 + 7 - s0, 8) * 8
            return pltpu.make_async_copy(
                la_hbm.at[pl.ds(h0, _NHW), pl.ds(p0x8, _BLK),
                          pl.ds(c2 * 128, _K)],
                st_ref.at[buf],
                sst.at[buf],
            )

        def out_cp(buf, qb):
            return pltpu.make_async_copy(
                st_ref.at[buf],
                out_hbm.at[pl.ds(h0, _NHW), pl.ds(qb, _BLK), :],
                sout.at[buf],
            )

        def qb_of(bb):
            return (jax.lax.div(q0, _BLK) + bb) * _BLK

        stage_cp(jnp.int32(0), qb_of(jnp.int32(0))).start()

        @pl.loop(0, _NBLK)
        def _(b):
            buf = jax.lax.rem(b, 3)
            bufn = jax.lax.rem(b + 1, 3)

            @pl.when(b >= 2)
            def _():
                out_cp(bufn, qb_of(b - 2)).wait()

            @pl.when(b + 1 < _NBLK)
            def _():
                stage_cp(bufn, qb_of(b + 1)).start()

            stage_cp(buf, qb_of(b)).wait()
            out_cp(buf, qb_of(b)).start()

        for b in (_NBLK - 2, _NBLK - 1):
            out_cp(jnp.int32(b % 3), qb_of(jnp.int32(b))).wait()

    return body(la)


def kernel(x, table):
    del x
    return _sc_expand(_build_la(table))
